# baseline (speedup 1.0000x reference)
import functools

import jax
import jax.numpy as jnp
from jax import lax
from jax.experimental import pallas as pl
from jax.experimental.pallas import tpu as pltpu
from jax.experimental.pallas import tpu_sc as plsc

N = 10000
E = 320000
DA = 128
DP = 16
H = 50
HP = 64
HP2 = 128
DX = 128

NC = 2
NS = 16
NW = NC * NS
L = 16

EPW = E // NW
C = 80
NCH = EPW // C
NP = 10240
RPS = NP // NS

_f32 = jnp.float32


def _tc_atoms(atom, Wx, bx, Waa, baa):
    BN = 1000

    def body(a_ref, wx_ref, bx_ref, waa_ref, baa_ref, x_ref, aa_ref):
        a = a_ref[...]
        x_ref[...] = jnp.dot(a, wx_ref[...], preferred_element_type=_f32) + bx_ref[...]
        aa_ref[...] = jnp.maximum(
            jnp.dot(a, waa_ref[...], preferred_element_type=_f32) + baa_ref[...], 0.0)

    return pl.pallas_call(
        body,
        grid=(N // BN,),
        in_specs=[
            pl.BlockSpec((BN, DA), lambda i: (i, 0)),
            pl.BlockSpec((DA, DX), lambda i: (0, 0)),
            pl.BlockSpec((1, DX), lambda i: (0, 0)),
            pl.BlockSpec((DA, H), lambda i: (0, 0)),
            pl.BlockSpec((1, H), lambda i: (0, 0)),
        ],
        out_specs=[
            pl.BlockSpec((BN, DX), lambda i: (i, 0)),
            pl.BlockSpec((BN, H), lambda i: (i, 0)),
        ],
        out_shape=[
            jax.ShapeDtypeStruct((N, DX), _f32),
            jax.ShapeDtypeStruct((N, H), _f32),
        ],
    )(atom, Wx, bx, Waa, baa)


def _tc_pae(pair, Wpa_p, bpa_p):
    BE = 2000

    def body(p_ref, w_ref, b_ref, o_ref):
        o_ref[...] = jnp.maximum(
            jnp.dot(p_ref[...], w_ref[...], preferred_element_type=_f32) + b_ref[...], 0.0)

    return pl.pallas_call(
        body,
        grid=(E // BE,),
        in_specs=[
            pl.BlockSpec((BE, DP), lambda i: (i, 0)),
            pl.BlockSpec((DP, HP2), lambda i: (0, 0)),
            pl.BlockSpec((1, HP2), lambda i: (0, 0)),
        ],
        out_specs=pl.BlockSpec((BE, HP2), lambda i: (i, 0)),
        out_shape=jax.ShapeDtypeStruct((E, HP2), _f32),
    )(pair, Wpa_p, bpa_p)


def _tc_p(S, pair, Wp1p, Wpp, bpp, Wp2, bp):
    BE = 2000

    def body(s_ref, pr_ref, w1_ref, wpp_ref, bpp_ref, w2_ref, bp_ref, o_ref):
        pp = jnp.maximum(
            jnp.dot(pr_ref[...], wpp_ref[...], preferred_element_type=_f32) + bpp_ref[...], 0.0)
        acc = jnp.dot(s_ref[...], w1_ref[...], preferred_element_type=_f32)
        acc = acc + jnp.dot(pp, w2_ref[...], preferred_element_type=_f32)
        o_ref[...] = jnp.maximum(acc + bp_ref[...], 0.0)

    return pl.pallas_call(
        body,
        grid=(E // BE,),
        in_specs=[
            pl.BlockSpec((BE, HP), lambda i: (i, 0)),
            pl.BlockSpec((BE, DP), lambda i: (i, 0)),
            pl.BlockSpec((HP, H), lambda i: (0, 0)),
            pl.BlockSpec((DP, H), lambda i: (0, 0)),
            pl.BlockSpec((1, H), lambda i: (0, 0)),
            pl.BlockSpec((H, H), lambda i: (0, 0)),
            pl.BlockSpec((1, H), lambda i: (0, 0)),
        ],
        out_specs=pl.BlockSpec((BE, H), lambda i: (i, 0)),
        out_shape=jax.ShapeDtypeStruct((E, H), _f32),
    )(S, pair, Wp1p, Wpp, bpp, Wp2, bp)


def _tc_a(AA, PAp, Wa1, Wa2p, ba):
    BN = 1000

    def body(aa_ref, pap_ref, w1_ref, w2_ref, b_ref, o_ref):
        pa = pap_ref[0] + pap_ref[1]
        acc = jnp.dot(aa_ref[...], w1_ref[...], preferred_element_type=_f32)
        acc = acc + jnp.dot(pa, w2_ref[...], preferred_element_type=_f32)
        o_ref[...] = jnp.maximum(acc + b_ref[...], 0.0)

    return pl.pallas_call(
        body,
        grid=(N // BN,),
        in_specs=[
            pl.BlockSpec((BN, H), lambda i: (i, 0)),
            pl.BlockSpec((NC, BN, HP2), lambda i: (0, i, 0)),
            pl.BlockSpec((H, H), lambda i: (0, 0)),
            pl.BlockSpec((HP2, H), lambda i: (0, 0)),
            pl.BlockSpec((1, H), lambda i: (0, 0)),
        ],
        out_specs=pl.BlockSpec((BN, H), lambda i: (i, 0)),
        out_shape=jax.ShapeDtypeStruct((N, H), _f32),
    )(AA, PAp, Wa1, Wa2p, ba)


def _sc_gather(X, idxi_r, idxj_r):
    mesh = plsc.VectorSubcoreMesh(core_axis_name="c", subcore_axis_name="s")
    TLAST = NCH - 1

    @functools.partial(
        pl.kernel,
        out_type=jax.ShapeDtypeStruct((NW, NCH, C, HP), _f32),
        mesh=mesh,
        scratch_types=[
            pltpu.VMEM((1, C), jnp.int32),
            pltpu.VMEM((1, C), jnp.int32),
            pltpu.VMEM((1, C), jnp.int32),
            pltpu.VMEM((1, C), jnp.int32),
            pltpu.VMEM((C, DX), _f32),
            pltpu.VMEM((C, DX), _f32),
            pltpu.VMEM((C, DX), _f32),
            pltpu.VMEM((C, DX), _f32),
            pltpu.VMEM((C, HP), _f32),
            pltpu.VMEM((C, HP), _f32),
            pltpu.SemaphoreType.DMA,
            pltpu.SemaphoreType.DMA,
            pltpu.SemaphoreType.DMA,
            pltpu.SemaphoreType.DMA,
            pltpu.SemaphoreType.DMA,
            pltpu.SemaphoreType.DMA,
        ],
    )
    def k(x_hbm, idxi_hbm, idxj_hbm, s_out,
          idxi_v0, idxi_v1, idxj_v0, idxj_v1, ri_v0, ri_v1, rj_v0, rj_v1,
          s_v0, s_v1,
          semg0, semg1, semi0, semi1, semo0, semo1):
        cid = lax.axis_index("c")
        sid = lax.axis_index("s")
        wid = sid * NC + cid
        idxi_v = (idxi_v0, idxi_v1)
        idxj_v = (idxj_v0, idxj_v1)
        ri_v = (ri_v0, ri_v1)
        rj_v = (rj_v0, rj_v1)
        s_v = (s_v0, s_v1)
        semg = (semg0, semg1)
        semi = (semi0, semi1)
        semo = (semo0, semo1)

        def issue_gathers(t, buf):
            pltpu.async_copy(x_hbm.at[idxi_v[buf].at[0]], ri_v[buf], semg[buf])
            pltpu.async_copy(x_hbm.at[idxj_v[buf].at[0]], rj_v[buf], semg[buf])

        def wait_gathers(buf):
            pltpu.make_async_copy(x_hbm.at[idxi_v[buf].at[0]], ri_v[buf], semg[buf]).wait()
            pltpu.make_async_copy(x_hbm.at[idxj_v[buf].at[0]], rj_v[buf], semg[buf]).wait()

        def issue_idx(t, buf):
            pltpu.async_copy(idxi_hbm.at[wid, pl.ds(t, 1)], idxi_v[buf], semi[buf])
            pltpu.async_copy(idxj_hbm.at[wid, pl.ds(t, 1)], idxj_v[buf], semi[buf])

        def wait_idx(t, buf):
            pltpu.make_async_copy(idxi_hbm.at[wid, pl.ds(t, 1)], idxi_v[buf], semi[buf]).wait()
            pltpu.make_async_copy(idxj_hbm.at[wid, pl.ds(t, 1)], idxj_v[buf], semi[buf]).wait()

        def compute(buf):
            rb, jb, sb = ri_v[buf], rj_v[buf], s_v[buf]

            def edge(e, c2):
                for k4 in range(HP // L):
                    c0 = k4 * L
                    t1 = jnp.maximum(
                        rb[e, pl.ds(c0, L)] + jb[e, pl.ds(HP + c0, L)], 0.0)
                    t2 = jnp.maximum(
                        rb[e, pl.ds(HP + c0, L)] + jb[e, pl.ds(c0, L)], 0.0)
                    sb[e, pl.ds(c0, L)] = t1 + t2
                return c2

            lax.fori_loop(0, C, edge, 0)

        issue_idx(0, 0)
        wait_idx(0, 0)
        issue_gathers(0, 0)
        issue_idx(1, 1)
        wait_idx(1, 1)

        def super_chunk(u, carry):
            for bb in range(2):
                t = 2 * u + bb
                nb = 1 - bb
                wait_gathers(bb)

                @pl.when(t + 1 <= TLAST)
                def _():
                    issue_gathers(t + 1, nb)

                @pl.when(t + 2 <= TLAST)
                def _():
                    issue_idx(t + 2, bb)

                @pl.when(t >= 2)
                def _():
                    pltpu.make_async_copy(
                        s_v[bb], s_out.at[wid, 0], semo[bb]).wait()

                compute(bb)
                pltpu.async_copy(s_v[bb], s_out.at[wid, t], semo[bb])

                @pl.when(t + 2 <= TLAST)
                def _():
                    wait_idx(t + 2, bb)
            return carry

        lax.fori_loop(0, NCH // 2, super_chunk, 0)

        t = TLAST
        wait_gathers(0)
        pltpu.make_async_copy(s_v[0], s_out.at[wid, 0], semo[0]).wait()
        compute(0)
        pltpu.sync_copy(s_v[0], s_out.at[wid, t])
        pltpu.make_async_copy(s_v[1], s_out.at[wid, 0], semo[1]).wait()

    return k(X, idxi_r, idxj_r)


def _sc_segsum(split_r, pae_r):
    mesh = plsc.VectorSubcoreMesh(core_axis_name="c", subcore_axis_name="s")
    TLAST = NCH - 1

    @functools.partial(
        pl.kernel,
        out_type=jax.ShapeDtypeStruct((NC, NP, HP2), _f32),
        mesh=mesh,
        scratch_types=[
            pltpu.VMEM((1, C), jnp.int32),
            pltpu.VMEM((1, C), jnp.int32),
            pltpu.VMEM((C, HP2), _f32),
            pltpu.VMEM((C, HP2), _f32),
            pltpu.VMEM((64, HP2), _f32),
            pltpu.VMEM_SHARED((NP, HP2), _f32),
            pltpu.SemaphoreType.DMA,
            pltpu.SemaphoreType.DMA,
            pltpu.SemaphoreType.DMA,
            pltpu.SemaphoreType.DMA,
        ],
    )
    def k(split_hbm, pae_hbm, pa_out, split_v0, split_v1, pae_v0, pae_v1,
          zbuf, shared, seml0, seml1, sems0, sems1):
        cid = lax.axis_index("c")
        sid = lax.axis_index("s")
        wid = sid * NC + cid
        split_v = (split_v0, split_v1)
        pae_v = (pae_v0, pae_v1)
        seml = (seml0, seml1)
        sems = (sems0, sems1)

        def zrow(r, carry):
            for k4 in range(HP2 // L):
                zbuf[r, pl.ds(k4 * L, L)] = jnp.zeros((L,), _f32)
            return carry

        lax.fori_loop(0, 64, zrow, 0)
        for u in range(RPS // 64):
            pltpu.sync_copy(zbuf, shared.at[pl.ds(sid * RPS + u * 64, 64)])
        plsc.subcore_barrier()

        def issue_loads(t, buf):
            pltpu.async_copy(split_hbm.at[wid, pl.ds(t, 1)], split_v[buf], seml[buf])
            pltpu.async_copy(pae_hbm.at[wid, t], pae_v[buf], seml[buf])

        def wait_loads(t, buf):
            pltpu.make_async_copy(split_hbm.at[wid, pl.ds(t, 1)], split_v[buf], seml[buf]).wait()
            pltpu.make_async_copy(pae_hbm.at[wid, t], pae_v[buf], seml[buf]).wait()

        def issue_scatter(buf):
            pltpu.async_copy(pae_v[buf], shared.at[split_v[buf].at[0]],
                             sems[buf], add=True)

        def drain_scatter(buf):
            pltpu.make_async_copy(pae_v[buf], shared.at[split_v[buf].at[0]],
                                  sems[buf]).wait()

        issue_loads(0, 0)
        issue_loads(1, 1)

        def super_chunk(u, carry):
            for bb in range(2):
                t = 2 * u + bb
                wait_loads(t, bb)
                issue_scatter(bb)

                @pl.when(t + 2 <= TLAST)
                def _():
                    drain_scatter(bb)
                    issue_loads(t + 2, bb)
            return carry

        lax.fori_loop(0, NCH // 2, super_chunk, 0)

        t = TLAST
        wait_loads(t, 0)
        issue_scatter(0)
        drain_scatter(1)
        drain_scatter(0)

        plsc.subcore_barrier()
        for u in range(RPS // 64):
            pltpu.sync_copy(shared.at[pl.ds(sid * RPS + u * 64, 64)], zbuf)
            pltpu.sync_copy(zbuf, pa_out.at[cid, pl.ds(sid * RPS + u * 64, 64)])

    return k(split_r, pae_r)


def kernel(atom_features, pair_features, pair_split, atom_to_pair,
           W_AA, b_AA, W_PA, b_PA, W_A, b_A,
           W_AP, b_AP, W_PP, b_PP, W_P, b_P):
    W1 = W_AP[:DA]
    W2 = W_AP[DA:]
    Wx = jnp.zeros((DA, DX), _f32).at[:, 0:H].set(W1).at[:, HP:HP + H].set(W2)
    bx = jnp.zeros((1, DX), _f32).at[0, 0:H].set(b_AP)
    Wpa_p = jnp.zeros((DP, HP2), _f32).at[:, :H].set(W_PA)
    bpa_p = jnp.zeros((1, HP2), _f32).at[0, :H].set(b_PA)
    Wp1p = jnp.zeros((HP, H), _f32).at[:H].set(W_P[:H])
    Wa2p = jnp.zeros((HP2, H), _f32).at[:H].set(W_A[H:])

    X, AA = _tc_atoms(atom_features, Wx, bx, W_AA, b_AA.reshape(1, H))
    PAe = _tc_pae(pair_features, Wpa_p, bpa_p)

    idxi_r = atom_to_pair[:, 0].reshape(NW, NCH, C)
    idxj_r = atom_to_pair[:, 1].reshape(NW, NCH, C)
    split_r = pair_split.reshape(NW, NCH, C)
    pae_r = PAe.reshape(NW, NCH, C, HP2)
    S_r = _sc_gather(X, idxi_r, idxj_r)
    split_r, pae_r, S_r = lax.optimization_barrier((split_r, pae_r, S_r))
    PAp = _sc_segsum(split_r, pae_r)
    S = S_r.reshape(E, HP)

    P = _tc_p(S, pair_features, Wp1p, W_PP, b_PP.reshape(1, H),
              W_P[H:], b_P.reshape(1, H))
    A = _tc_a(AA, PAp, W_A[:H], Wa2p, b_A.reshape(1, H))
    return (A, P)


# speedup vs baseline: 1.3203x; 1.3203x over previous
import functools

import jax
import jax.numpy as jnp
from jax import lax
from jax.experimental import pallas as pl
from jax.experimental.pallas import tpu as pltpu
from jax.experimental.pallas import tpu_sc as plsc

N = 10000
E = 320000
DA = 128
DP = 16
H = 50
HP = 64
HP2 = 128
DX = 128

NC = 2
NS = 16
NW = NC * NS
L = 16

EPW = E // NW
C = 80
NCH = EPW // C
NP = 10240
RPS = NP // NS

_f32 = jnp.float32


def _tc_atoms(atom, Wx, bx, Waa, baa):
    BN = 1000

    def body(a_ref, wx_ref, bx_ref, waa_ref, baa_ref, x_ref, aa_ref):
        a = a_ref[...]
        x_ref[...] = jnp.dot(a, wx_ref[...], preferred_element_type=_f32) + bx_ref[...]
        aa_ref[...] = jnp.maximum(
            jnp.dot(a, waa_ref[...], preferred_element_type=_f32) + baa_ref[...], 0.0)

    return pl.pallas_call(
        body,
        grid=(N // BN,),
        in_specs=[
            pl.BlockSpec((BN, DA), lambda i: (i, 0)),
            pl.BlockSpec((DA, DX), lambda i: (0, 0)),
            pl.BlockSpec((1, DX), lambda i: (0, 0)),
            pl.BlockSpec((DA, H), lambda i: (0, 0)),
            pl.BlockSpec((1, H), lambda i: (0, 0)),
        ],
        out_specs=[
            pl.BlockSpec((BN, DX), lambda i: (i, 0)),
            pl.BlockSpec((BN, H), lambda i: (i, 0)),
        ],
        out_shape=[
            jax.ShapeDtypeStruct((N, DX), _f32),
            jax.ShapeDtypeStruct((N, H), _f32),
        ],
    )(atom, Wx, bx, Waa, baa)


def _tc_pae(pair, Wpa_p, bpa_p):
    BE = 2000

    def body(p_ref, w_ref, b_ref, o_ref):
        o_ref[...] = jnp.maximum(
            jnp.dot(p_ref[...], w_ref[...], preferred_element_type=_f32) + b_ref[...], 0.0)

    return pl.pallas_call(
        body,
        grid=(E // BE,),
        in_specs=[
            pl.BlockSpec((BE, DP), lambda i: (i, 0)),
            pl.BlockSpec((DP, HP2), lambda i: (0, 0)),
            pl.BlockSpec((1, HP2), lambda i: (0, 0)),
        ],
        out_specs=pl.BlockSpec((BE, HP2), lambda i: (i, 0)),
        out_shape=jax.ShapeDtypeStruct((E, HP2), _f32),
    )(pair, Wpa_p, bpa_p)


def _tc_p(S, pair_t, Wp1p, Wpp, bpp, Wp2, bp):
    BE = 2560
    dn_t = (((0,), (0,)), ((), ()))

    def body(s_ref, pr_ref, w1_ref, wpp_ref, bpp_ref, w2_ref, bp_ref, o_ref):
        pp_t = jnp.maximum(
            lax.dot_general(wpp_ref[...], pr_ref[...], dn_t,
                            preferred_element_type=_f32) + bpp_ref[...], 0.0)
        acc = lax.dot_general(w1_ref[...], s_ref[...], (((0,), (1,)), ((), ())),
                              preferred_element_type=_f32)
        acc = acc + lax.dot_general(w2_ref[...], pp_t, (((0,), (0,)), ((), ())),
                                    preferred_element_type=_f32)
        o_ref[...] = jnp.maximum(acc + bp_ref[...], 0.0)

    return pl.pallas_call(
        body,
        grid=(E // BE,),
        in_specs=[
            pl.BlockSpec((BE, HP), lambda i: (i, 0)),
            pl.BlockSpec((DP, BE), lambda i: (0, i)),
            pl.BlockSpec((HP, H), lambda i: (0, 0)),
            pl.BlockSpec((DP, H), lambda i: (0, 0)),
            pl.BlockSpec((H, 1), lambda i: (0, 0)),
            pl.BlockSpec((H, H), lambda i: (0, 0)),
            pl.BlockSpec((H, 1), lambda i: (0, 0)),
        ],
        out_specs=pl.BlockSpec((H, BE), lambda i: (0, i)),
        out_shape=jax.ShapeDtypeStruct((H, E), _f32),
    )(S, pair_t, Wp1p, Wpp, bpp, Wp2, bp)


def _tc_a(AA, PAp, Wa1, Wa2p, ba):
    BN = 1000

    def body(aa_ref, pap_ref, w1_ref, w2_ref, b_ref, o_ref):
        pa = pap_ref[0] + pap_ref[1]
        acc = jnp.dot(aa_ref[...], w1_ref[...], preferred_element_type=_f32)
        acc = acc + jnp.dot(pa, w2_ref[...], preferred_element_type=_f32)
        o_ref[...] = jnp.maximum(acc + b_ref[...], 0.0)

    return pl.pallas_call(
        body,
        grid=(N // BN,),
        in_specs=[
            pl.BlockSpec((BN, H), lambda i: (i, 0)),
            pl.BlockSpec((NC, BN, HP2), lambda i: (0, i, 0)),
            pl.BlockSpec((H, H), lambda i: (0, 0)),
            pl.BlockSpec((HP2, H), lambda i: (0, 0)),
            pl.BlockSpec((1, H), lambda i: (0, 0)),
        ],
        out_specs=pl.BlockSpec((BN, H), lambda i: (i, 0)),
        out_shape=jax.ShapeDtypeStruct((N, H), _f32),
    )(AA, PAp, Wa1, Wa2p, ba)


def _sc_gather(X, idxi_r, idxj_r):
    mesh = plsc.VectorSubcoreMesh(core_axis_name="c", subcore_axis_name="s")
    TLAST = NCH - 1

    @functools.partial(
        pl.kernel,
        out_type=jax.ShapeDtypeStruct((NW, NCH, C, HP), _f32),
        mesh=mesh,
        scratch_types=[
            pltpu.VMEM((1, C), jnp.int32),
            pltpu.VMEM((1, C), jnp.int32),
            pltpu.VMEM((1, C), jnp.int32),
            pltpu.VMEM((1, C), jnp.int32),
            pltpu.VMEM((C, DX), _f32),
            pltpu.VMEM((C, DX), _f32),
            pltpu.VMEM((C, DX), _f32),
            pltpu.VMEM((C, DX), _f32),
            pltpu.VMEM((C, HP), _f32),
            pltpu.VMEM((C, HP), _f32),
            pltpu.SemaphoreType.DMA,
            pltpu.SemaphoreType.DMA,
            pltpu.SemaphoreType.DMA,
            pltpu.SemaphoreType.DMA,
            pltpu.SemaphoreType.DMA,
            pltpu.SemaphoreType.DMA,
        ],
    )
    def k(x_hbm, idxi_hbm, idxj_hbm, s_out,
          idxi_v0, idxi_v1, idxj_v0, idxj_v1, ri_v0, ri_v1, rj_v0, rj_v1,
          s_v0, s_v1,
          semg0, semg1, semi0, semi1, semo0, semo1):
        cid = lax.axis_index("c")
        sid = lax.axis_index("s")
        wid = sid * NC + cid
        idxi_v = (idxi_v0, idxi_v1)
        idxj_v = (idxj_v0, idxj_v1)
        ri_v = (ri_v0, ri_v1)
        rj_v = (rj_v0, rj_v1)
        s_v = (s_v0, s_v1)
        semg = (semg0, semg1)
        semi = (semi0, semi1)
        semo = (semo0, semo1)

        def issue_gathers(t, buf):
            pltpu.async_copy(x_hbm.at[idxi_v[buf].at[0]], ri_v[buf], semg[buf])
            pltpu.async_copy(x_hbm.at[idxj_v[buf].at[0]], rj_v[buf], semg[buf])

        def wait_gathers(buf):
            pltpu.make_async_copy(x_hbm.at[idxi_v[buf].at[0]], ri_v[buf], semg[buf]).wait()
            pltpu.make_async_copy(x_hbm.at[idxj_v[buf].at[0]], rj_v[buf], semg[buf]).wait()

        def issue_idx(t, buf):
            pltpu.async_copy(idxi_hbm.at[wid, pl.ds(t, 1)], idxi_v[buf], semi[buf])
            pltpu.async_copy(idxj_hbm.at[wid, pl.ds(t, 1)], idxj_v[buf], semi[buf])

        def wait_idx(t, buf):
            pltpu.make_async_copy(idxi_hbm.at[wid, pl.ds(t, 1)], idxi_v[buf], semi[buf]).wait()
            pltpu.make_async_copy(idxj_hbm.at[wid, pl.ds(t, 1)], idxj_v[buf], semi[buf]).wait()

        def compute(buf):
            rb, jb, sb = ri_v[buf], rj_v[buf], s_v[buf]

            def edge(e, c2):
                for k4 in range(HP // L):
                    c0 = k4 * L
                    t1 = jnp.maximum(
                        rb[e, pl.ds(c0, L)] + jb[e, pl.ds(HP + c0, L)], 0.0)
                    t2 = jnp.maximum(
                        rb[e, pl.ds(HP + c0, L)] + jb[e, pl.ds(c0, L)], 0.0)
                    sb[e, pl.ds(c0, L)] = t1 + t2
                return c2

            lax.fori_loop(0, C, edge, 0)

        issue_idx(0, 0)
        wait_idx(0, 0)
        issue_gathers(0, 0)
        issue_idx(1, 1)
        wait_idx(1, 1)

        def super_chunk(u, carry):
            for bb in range(2):
                t = 2 * u + bb
                nb = 1 - bb
                wait_gathers(bb)

                @pl.when(t + 1 <= TLAST)
                def _():
                    issue_gathers(t + 1, nb)

                @pl.when(t + 2 <= TLAST)
                def _():
                    issue_idx(t + 2, bb)

                @pl.when(t >= 2)
                def _():
                    pltpu.make_async_copy(
                        s_v[bb], s_out.at[wid, 0], semo[bb]).wait()

                compute(bb)
                pltpu.async_copy(s_v[bb], s_out.at[wid, t], semo[bb])

                @pl.when(t + 2 <= TLAST)
                def _():
                    wait_idx(t + 2, bb)
            return carry

        lax.fori_loop(0, NCH // 2, super_chunk, 0)

        t = TLAST
        wait_gathers(0)
        pltpu.make_async_copy(s_v[0], s_out.at[wid, 0], semo[0]).wait()
        compute(0)
        pltpu.sync_copy(s_v[0], s_out.at[wid, t])
        pltpu.make_async_copy(s_v[1], s_out.at[wid, 0], semo[1]).wait()

    return k(X, idxi_r, idxj_r)


def _sc_segsum(split_r, pae_r):
    mesh = plsc.VectorSubcoreMesh(core_axis_name="c", subcore_axis_name="s")
    TLAST = NCH - 1

    @functools.partial(
        pl.kernel,
        out_type=jax.ShapeDtypeStruct((NC, NP, HP2), _f32),
        mesh=mesh,
        scratch_types=[
            pltpu.VMEM((1, C), jnp.int32),
            pltpu.VMEM((1, C), jnp.int32),
            pltpu.VMEM((C, HP2), _f32),
            pltpu.VMEM((C, HP2), _f32),
            pltpu.VMEM((64, HP2), _f32),
            pltpu.VMEM_SHARED((NP, HP2), _f32),
            pltpu.SemaphoreType.DMA,
            pltpu.SemaphoreType.DMA,
            pltpu.SemaphoreType.DMA,
            pltpu.SemaphoreType.DMA,
        ],
    )
    def k(split_hbm, pae_hbm, pa_out, split_v0, split_v1, pae_v0, pae_v1,
          zbuf, shared, seml0, seml1, sems0, sems1):
        cid = lax.axis_index("c")
        sid = lax.axis_index("s")
        wid = sid * NC + cid
        split_v = (split_v0, split_v1)
        pae_v = (pae_v0, pae_v1)
        seml = (seml0, seml1)
        sems = (sems0, sems1)

        def zrow(r, carry):
            for k4 in range(HP2 // L):
                zbuf[r, pl.ds(k4 * L, L)] = jnp.zeros((L,), _f32)
            return carry

        lax.fori_loop(0, 64, zrow, 0)
        for u in range(RPS // 64):
            pltpu.sync_copy(zbuf, shared.at[pl.ds(sid * RPS + u * 64, 64)])
        plsc.subcore_barrier()

        def issue_loads(t, buf):
            pltpu.async_copy(split_hbm.at[wid, pl.ds(t, 1)], split_v[buf], seml[buf])
            pltpu.async_copy(pae_hbm.at[wid, t], pae_v[buf], seml[buf])

        def wait_loads(t, buf):
            pltpu.make_async_copy(split_hbm.at[wid, pl.ds(t, 1)], split_v[buf], seml[buf]).wait()
            pltpu.make_async_copy(pae_hbm.at[wid, t], pae_v[buf], seml[buf]).wait()

        def issue_scatter(buf):
            pltpu.async_copy(pae_v[buf], shared.at[split_v[buf].at[0]],
                             sems[buf], add=True)

        def drain_scatter(buf):
            pltpu.make_async_copy(pae_v[buf], shared.at[split_v[buf].at[0]],
                                  sems[buf]).wait()

        issue_loads(0, 0)
        issue_loads(1, 1)

        def super_chunk(u, carry):
            for bb in range(2):
                t = 2 * u + bb
                wait_loads(t, bb)
                issue_scatter(bb)

                @pl.when(t + 2 <= TLAST)
                def _():
                    drain_scatter(bb)
                    issue_loads(t + 2, bb)
            return carry

        lax.fori_loop(0, NCH // 2, super_chunk, 0)

        t = TLAST
        wait_loads(t, 0)
        issue_scatter(0)
        drain_scatter(1)
        drain_scatter(0)

        plsc.subcore_barrier()
        for u in range(RPS // 64):
            pltpu.sync_copy(shared.at[pl.ds(sid * RPS + u * 64, 64)], zbuf)
            pltpu.sync_copy(zbuf, pa_out.at[cid, pl.ds(sid * RPS + u * 64, 64)])

    return k(split_r, pae_r)


def kernel(atom_features, pair_features, pair_split, atom_to_pair,
           W_AA, b_AA, W_PA, b_PA, W_A, b_A,
           W_AP, b_AP, W_PP, b_PP, W_P, b_P):
    W1 = W_AP[:DA]
    W2 = W_AP[DA:]
    Wx = jnp.zeros((DA, DX), _f32).at[:, 0:H].set(W1).at[:, HP:HP + H].set(W2)
    bx = jnp.zeros((1, DX), _f32).at[0, 0:H].set(b_AP)
    Wpa_p = jnp.zeros((DP, HP2), _f32).at[:, :H].set(W_PA)
    bpa_p = jnp.zeros((1, HP2), _f32).at[0, :H].set(b_PA)
    Wp1p = jnp.zeros((HP, H), _f32).at[:H].set(W_P[:H])
    Wa2p = jnp.zeros((HP2, H), _f32).at[:H].set(W_A[H:])

    X, AA = _tc_atoms(atom_features, Wx, bx, W_AA, b_AA.reshape(1, H))
    PAe = _tc_pae(pair_features, Wpa_p, bpa_p)

    idxi_r = atom_to_pair[:, 0].reshape(NW, NCH, C)
    idxj_r = atom_to_pair[:, 1].reshape(NW, NCH, C)
    split_r = pair_split.reshape(NW, NCH, C)
    pae_r = PAe.reshape(NW, NCH, C, HP2)
    S_r = _sc_gather(X, idxi_r, idxj_r)
    split_r, pae_r, S_r = lax.optimization_barrier((split_r, pae_r, S_r))
    PAp = _sc_segsum(split_r, pae_r)
    S = S_r.reshape(E, HP)

    P_t = _tc_p(S, pair_features.T, Wp1p, W_PP, b_PP.reshape(H, 1),
                W_P[H:], b_P.reshape(H, 1))
    A = _tc_a(AA, PAp, W_A[:H], Wa2p, b_A.reshape(1, H))
    return (A, P_t.T)


# speedup vs baseline: 1.3308x; 1.0080x over previous
import functools

import jax
import jax.numpy as jnp
from jax import lax
from jax.experimental import pallas as pl
from jax.experimental.pallas import tpu as pltpu
from jax.experimental.pallas import tpu_sc as plsc

N = 10000
E = 320000
DA = 128
DP = 16
H = 50
HP = 64
HP2 = 128
DX = 128

NC = 2
NS = 16
NW = NC * NS
L = 16

EPW = E // NW
C = 80
NCH = EPW // C
NP = 10240
RPS = NP // NS

_f32 = jnp.float32


def _tc_atoms(atom, Wx, bx, Waa, baa):
    BN = 1000

    def body(a_ref, wx_ref, bx_ref, waa_ref, baa_ref, x_ref, aa_ref):
        a = a_ref[...]
        x_ref[...] = jnp.dot(a, wx_ref[...], preferred_element_type=_f32) + bx_ref[...]
        aa_ref[...] = jnp.maximum(
            jnp.dot(a, waa_ref[...], preferred_element_type=_f32) + baa_ref[...], 0.0)

    return pl.pallas_call(
        body,
        grid=(N // BN,),
        in_specs=[
            pl.BlockSpec((BN, DA), lambda i: (i, 0)),
            pl.BlockSpec((DA, DX), lambda i: (0, 0)),
            pl.BlockSpec((1, DX), lambda i: (0, 0)),
            pl.BlockSpec((DA, H), lambda i: (0, 0)),
            pl.BlockSpec((1, H), lambda i: (0, 0)),
        ],
        out_specs=[
            pl.BlockSpec((BN, DX), lambda i: (i, 0)),
            pl.BlockSpec((BN, H), lambda i: (i, 0)),
        ],
        out_shape=[
            jax.ShapeDtypeStruct((N, DX), _f32),
            jax.ShapeDtypeStruct((N, H), _f32),
        ],
    )(atom, Wx, bx, Waa, baa)


def _tc_pae(pair, Wpa_p, bpa_p):
    BE = 2000

    def body(p_ref, w_ref, b_ref, o_ref):
        o_ref[...] = jnp.maximum(
            jnp.dot(p_ref[...], w_ref[...], preferred_element_type=_f32) + b_ref[...], 0.0)

    return pl.pallas_call(
        body,
        grid=(E // BE,),
        in_specs=[
            pl.BlockSpec((BE, DP), lambda i: (i, 0)),
            pl.BlockSpec((DP, HP2), lambda i: (0, 0)),
            pl.BlockSpec((1, HP2), lambda i: (0, 0)),
        ],
        out_specs=pl.BlockSpec((BE, HP2), lambda i: (i, 0)),
        out_shape=jax.ShapeDtypeStruct((E, HP2), _f32),
    )(pair, Wpa_p, bpa_p)


def _tc_p(S, pair_t, Wp1p, Wpp, bpp, Wp2, bp):
    BE = 2560
    dn_t = (((0,), (0,)), ((), ()))

    def body(s_ref, pr_ref, w1_ref, wpp_ref, bpp_ref, w2_ref, bp_ref, o_ref):
        pp_t = jnp.maximum(
            lax.dot_general(wpp_ref[...], pr_ref[...], dn_t,
                            preferred_element_type=_f32) + bpp_ref[...], 0.0)
        acc = lax.dot_general(w1_ref[...], s_ref[...], (((0,), (1,)), ((), ())),
                              preferred_element_type=_f32)
        acc = acc + lax.dot_general(w2_ref[...], pp_t, (((0,), (0,)), ((), ())),
                                    preferred_element_type=_f32)
        o_ref[...] = jnp.maximum(acc + bp_ref[...], 0.0)

    return pl.pallas_call(
        body,
        grid=(E // BE,),
        in_specs=[
            pl.BlockSpec((BE, HP), lambda i: (i, 0)),
            pl.BlockSpec((DP, BE), lambda i: (0, i)),
            pl.BlockSpec((HP, H), lambda i: (0, 0)),
            pl.BlockSpec((DP, H), lambda i: (0, 0)),
            pl.BlockSpec((H, 1), lambda i: (0, 0)),
            pl.BlockSpec((H, H), lambda i: (0, 0)),
            pl.BlockSpec((H, 1), lambda i: (0, 0)),
        ],
        out_specs=pl.BlockSpec((H, BE), lambda i: (0, i)),
        out_shape=jax.ShapeDtypeStruct((H, E), _f32),
    )(S, pair_t, Wp1p, Wpp, bpp, Wp2, bp)


def _tc_a(AA, PAp, Wa1, Wa2p, ba):
    BN = 1000

    def body(aa_ref, pap_ref, w1_ref, w2_ref, b_ref, o_ref):
        pa = pap_ref[0] + pap_ref[1]
        acc = jnp.dot(aa_ref[...], w1_ref[...], preferred_element_type=_f32)
        acc = acc + jnp.dot(pa, w2_ref[...], preferred_element_type=_f32)
        o_ref[...] = jnp.maximum(acc + b_ref[...], 0.0)

    return pl.pallas_call(
        body,
        grid=(N // BN,),
        in_specs=[
            pl.BlockSpec((BN, H), lambda i: (i, 0)),
            pl.BlockSpec((NC, BN, HP2), lambda i: (0, i, 0)),
            pl.BlockSpec((H, H), lambda i: (0, 0)),
            pl.BlockSpec((HP2, H), lambda i: (0, 0)),
            pl.BlockSpec((1, H), lambda i: (0, 0)),
        ],
        out_specs=pl.BlockSpec((BN, H), lambda i: (i, 0)),
        out_shape=jax.ShapeDtypeStruct((N, H), _f32),
    )(AA, PAp, Wa1, Wa2p, ba)


def _sc_gather(X, idxi_r, idxj_r):
    mesh = plsc.VectorSubcoreMesh(core_axis_name="c", subcore_axis_name="s")
    TLAST = NCH - 1

    @functools.partial(
        pl.kernel,
        out_type=jax.ShapeDtypeStruct((NW, NCH, C, HP), _f32),
        mesh=mesh,
        scratch_types=[
            pltpu.VMEM((1, C), jnp.int32),
            pltpu.VMEM((1, C), jnp.int32),
            pltpu.VMEM((1, C), jnp.int32),
            pltpu.VMEM((1, C), jnp.int32),
            pltpu.VMEM((C, DX), _f32),
            pltpu.VMEM((C, DX), _f32),
            pltpu.VMEM((C, DX), _f32),
            pltpu.VMEM((C, DX), _f32),
            pltpu.VMEM((C, HP), _f32),
            pltpu.VMEM((C, HP), _f32),
            pltpu.SemaphoreType.DMA,
            pltpu.SemaphoreType.DMA,
            pltpu.SemaphoreType.DMA,
            pltpu.SemaphoreType.DMA,
            pltpu.SemaphoreType.DMA,
            pltpu.SemaphoreType.DMA,
        ],
    )
    def k(x_hbm, idxi_hbm, idxj_hbm, s_out,
          idxi_v0, idxi_v1, idxj_v0, idxj_v1, ri_v0, ri_v1, rj_v0, rj_v1,
          s_v0, s_v1,
          semg0, semg1, semi0, semi1, semo0, semo1):
        cid = lax.axis_index("c")
        sid = lax.axis_index("s")
        wid = sid * NC + cid
        idxi_v = (idxi_v0, idxi_v1)
        idxj_v = (idxj_v0, idxj_v1)
        ri_v = (ri_v0, ri_v1)
        rj_v = (rj_v0, rj_v1)
        s_v = (s_v0, s_v1)
        semg = (semg0, semg1)
        semi = (semi0, semi1)
        semo = (semo0, semo1)

        def issue_gathers(t, buf):
            pltpu.async_copy(x_hbm.at[idxi_v[buf].at[0]], ri_v[buf], semg[buf])
            pltpu.async_copy(x_hbm.at[idxj_v[buf].at[0]], rj_v[buf], semg[buf])

        def wait_gathers(buf):
            pltpu.make_async_copy(x_hbm.at[idxi_v[buf].at[0]], ri_v[buf], semg[buf]).wait()
            pltpu.make_async_copy(x_hbm.at[idxj_v[buf].at[0]], rj_v[buf], semg[buf]).wait()

        def issue_idx(t, buf):
            pltpu.async_copy(idxi_hbm.at[wid, pl.ds(t, 1)], idxi_v[buf], semi[buf])
            pltpu.async_copy(idxj_hbm.at[wid, pl.ds(t, 1)], idxj_v[buf], semi[buf])

        def wait_idx(t, buf):
            pltpu.make_async_copy(idxi_hbm.at[wid, pl.ds(t, 1)], idxi_v[buf], semi[buf]).wait()
            pltpu.make_async_copy(idxj_hbm.at[wid, pl.ds(t, 1)], idxj_v[buf], semi[buf]).wait()

        def compute(buf):
            rb, jb, sb = ri_v[buf], rj_v[buf], s_v[buf]

            def edge(e, c2):
                for k4 in range(HP // L):
                    c0 = k4 * L
                    t1 = jnp.maximum(
                        rb[e, pl.ds(c0, L)] + jb[e, pl.ds(HP + c0, L)], 0.0)
                    t2 = jnp.maximum(
                        rb[e, pl.ds(HP + c0, L)] + jb[e, pl.ds(c0, L)], 0.0)
                    sb[e, pl.ds(c0, L)] = t1 + t2
                return c2

            lax.fori_loop(0, C, edge, 0, unroll=4)

        issue_idx(0, 0)
        wait_idx(0, 0)
        issue_gathers(0, 0)
        issue_idx(1, 1)
        wait_idx(1, 1)

        def super_chunk(u, carry):
            for bb in range(2):
                t = 2 * u + bb
                nb = 1 - bb
                wait_gathers(bb)

                @pl.when(t + 1 <= TLAST)
                def _():
                    issue_gathers(t + 1, nb)

                @pl.when(t + 2 <= TLAST)
                def _():
                    issue_idx(t + 2, bb)

                @pl.when(t >= 2)
                def _():
                    pltpu.make_async_copy(
                        s_v[bb], s_out.at[wid, 0], semo[bb]).wait()

                compute(bb)
                pltpu.async_copy(s_v[bb], s_out.at[wid, t], semo[bb])

                @pl.when(t + 2 <= TLAST)
                def _():
                    wait_idx(t + 2, bb)
            return carry

        lax.fori_loop(0, NCH // 2, super_chunk, 0)

        t = TLAST
        wait_gathers(0)
        pltpu.make_async_copy(s_v[0], s_out.at[wid, 0], semo[0]).wait()
        compute(0)
        pltpu.sync_copy(s_v[0], s_out.at[wid, t])
        pltpu.make_async_copy(s_v[1], s_out.at[wid, 0], semo[1]).wait()

    return k(X, idxi_r, idxj_r)


def _sc_segsum(split_r, pae_r):
    mesh = plsc.VectorSubcoreMesh(core_axis_name="c", subcore_axis_name="s")
    TLAST = NCH - 1

    @functools.partial(
        pl.kernel,
        out_type=jax.ShapeDtypeStruct((NC, NP, HP2), _f32),
        mesh=mesh,
        scratch_types=[
            pltpu.VMEM((1, C), jnp.int32),
            pltpu.VMEM((1, C), jnp.int32),
            pltpu.VMEM((C, HP2), _f32),
            pltpu.VMEM((C, HP2), _f32),
            pltpu.VMEM((64, HP2), _f32),
            pltpu.VMEM_SHARED((NP, HP2), _f32),
            pltpu.SemaphoreType.DMA,
            pltpu.SemaphoreType.DMA,
            pltpu.SemaphoreType.DMA,
            pltpu.SemaphoreType.DMA,
        ],
    )
    def k(split_hbm, pae_hbm, pa_out, split_v0, split_v1, pae_v0, pae_v1,
          zbuf, shared, seml0, seml1, sems0, sems1):
        cid = lax.axis_index("c")
        sid = lax.axis_index("s")
        wid = sid * NC + cid
        split_v = (split_v0, split_v1)
        pae_v = (pae_v0, pae_v1)
        seml = (seml0, seml1)
        sems = (sems0, sems1)

        def zrow(r, carry):
            for k4 in range(HP2 // L):
                zbuf[r, pl.ds(k4 * L, L)] = jnp.zeros((L,), _f32)
            return carry

        lax.fori_loop(0, 64, zrow, 0)
        for u in range(RPS // 64):
            pltpu.sync_copy(zbuf, shared.at[pl.ds(sid * RPS + u * 64, 64)])
        plsc.subcore_barrier()

        def issue_loads(t, buf):
            pltpu.async_copy(split_hbm.at[wid, pl.ds(t, 1)], split_v[buf], seml[buf])
            pltpu.async_copy(pae_hbm.at[wid, t], pae_v[buf], seml[buf])

        def wait_loads(t, buf):
            pltpu.make_async_copy(split_hbm.at[wid, pl.ds(t, 1)], split_v[buf], seml[buf]).wait()
            pltpu.make_async_copy(pae_hbm.at[wid, t], pae_v[buf], seml[buf]).wait()

        def issue_scatter(buf):
            pltpu.async_copy(pae_v[buf], shared.at[split_v[buf].at[0]],
                             sems[buf], add=True)

        def drain_scatter(buf):
            pltpu.make_async_copy(pae_v[buf], shared.at[split_v[buf].at[0]],
                                  sems[buf]).wait()

        issue_loads(0, 0)
        issue_loads(1, 1)

        def super_chunk(u, carry):
            for bb in range(2):
                t = 2 * u + bb
                wait_loads(t, bb)
                issue_scatter(bb)

                @pl.when(t + 2 <= TLAST)
                def _():
                    drain_scatter(bb)
                    issue_loads(t + 2, bb)
            return carry

        lax.fori_loop(0, NCH // 2, super_chunk, 0)

        t = TLAST
        wait_loads(t, 0)
        issue_scatter(0)
        drain_scatter(1)
        drain_scatter(0)

        plsc.subcore_barrier()
        for u in range(RPS // 64):
            pltpu.sync_copy(shared.at[pl.ds(sid * RPS + u * 64, 64)], zbuf)
            pltpu.sync_copy(zbuf, pa_out.at[cid, pl.ds(sid * RPS + u * 64, 64)])

    return k(split_r, pae_r)


def kernel(atom_features, pair_features, pair_split, atom_to_pair,
           W_AA, b_AA, W_PA, b_PA, W_A, b_A,
           W_AP, b_AP, W_PP, b_PP, W_P, b_P):
    W1 = W_AP[:DA]
    W2 = W_AP[DA:]
    Wx = jnp.zeros((DA, DX), _f32).at[:, 0:H].set(W1).at[:, HP:HP + H].set(W2)
    bx = jnp.zeros((1, DX), _f32).at[0, 0:H].set(b_AP)
    Wpa_p = jnp.zeros((DP, HP2), _f32).at[:, :H].set(W_PA)
    bpa_p = jnp.zeros((1, HP2), _f32).at[0, :H].set(b_PA)
    Wp1p = jnp.zeros((HP, H), _f32).at[:H].set(W_P[:H])
    Wa2p = jnp.zeros((HP2, H), _f32).at[:H].set(W_A[H:])

    X, AA = _tc_atoms(atom_features, Wx, bx, W_AA, b_AA.reshape(1, H))
    PAe = _tc_pae(pair_features, Wpa_p, bpa_p)

    idxi_r = atom_to_pair[:, 0].reshape(NW, NCH, C)
    idxj_r = atom_to_pair[:, 1].reshape(NW, NCH, C)
    split_r = pair_split.reshape(NW, NCH, C)
    pae_r = PAe.reshape(NW, NCH, C, HP2)
    S_r = _sc_gather(X, idxi_r, idxj_r)
    split_r, pae_r, S_r = lax.optimization_barrier((split_r, pae_r, S_r))
    PAp = _sc_segsum(split_r, pae_r)
    S = S_r.reshape(E, HP)

    P_t = _tc_p(S, pair_features.T, Wp1p, W_PP, b_PP.reshape(H, 1),
                W_P[H:], b_P.reshape(H, 1))
    A = _tc_a(AA, PAp, W_A[:H], Wa2p, b_A.reshape(1, H))
    return (A, P_t.T)
